# baseline (speedup 1.0000x reference)
import functools

import jax
import jax.numpy as jnp
from jax import lax
from jax.experimental import pallas as pl
from jax.experimental.pallas import tpu as pltpu

K = 32


def _sortable_key(f):
    b = lax.bitcast_convert_type(f, jnp.int32)
    return jnp.where(b < 0, jnp.bitwise_xor(b, jnp.int32(0x7FFFFFFF)), b)


def _key_to_f32(k):
    b = jnp.where(k < 0, jnp.bitwise_xor(k, jnp.int32(0x7FFFFFFF)), k)
    return lax.bitcast_convert_type(b, jnp.float32)


def _mid(lo, hi):
    return (lo >> 1) + (hi >> 1) + (lo & hi & 1)


def _bisect_step(l, lo, hi):
    mid = _mid(lo, hi)
    fm = _key_to_f32(mid)
    cnt = jnp.sum((l >= fm).astype(jnp.int32), axis=1, keepdims=True)
    gap = lax.bitcast_convert_type(hi - lo, jnp.uint32)
    active = gap > 1
    ge = cnt >= K
    eq = cnt == K
    nlo = jnp.where(ge, mid, lo)
    nhi = jnp.where(ge, hi, mid)
    nhi = jnp.where(eq, nlo + 1, nhi)
    lo = jnp.where(active, nlo, lo)
    hi = jnp.where(active, nhi, hi)
    return lo, hi


def _kernel_body(ni, num_g, gt, temp_ref, x_ref, w_ref, out_ref,
                 acc_ref, lo_ref, hi_ref, m_ref, zinv_ref):
    i = pl.program_id(0)
    j = pl.program_id(1)

    @pl.when(i >= 2)
    def _drain():
        sl = lax.rem(i, 2)
        e_col = acc_ref[sl, :, pl.ds(j * gt, gt)]
        out_ref[...] = e_col * zinv_ref[sl, :, :]

    @pl.when(i < ni)
    def _fill():
        sl = lax.rem(i, 2)
        logits = lax.dot_general(
            x_ref[...], w_ref[...], (((1,), (1,)), ((), ())),
            preferred_element_type=jnp.float32,
        ) / temp_ref[0]
        acc_ref[sl, :, pl.ds(j * gt, gt)] = logits

    @pl.when(jnp.logical_and(i >= 1, i <= ni))
    def _search():
        sl = lax.rem(i - 1, 2)
        rb = acc_ref.shape[1]
        gfull = acc_ref.shape[2]
        strip = min(128, rb)
        for r0 in range(0, rb, strip):
            l = acc_ref[sl, pl.ds(r0, strip), :]

            @pl.when(j == 0)
            def _init(l=l, r0=r0):
                m = jnp.max(l, axis=1, keepdims=True)
                cw = gfull // K
                s = m
                for c in range(K):
                    s = jnp.minimum(
                        s, jnp.max(l[:, c * cw:(c + 1) * cw], axis=1,
                                   keepdims=True))
                m_ref[sl, pl.ds(r0, strip)] = m
                lo_ref[sl, pl.ds(r0, strip)] = _sortable_key(s)
                hi_ref[sl, pl.ds(r0, strip)] = _sortable_key(m) + 1

            lo = lo_ref[sl, pl.ds(r0, strip)]
            hi = hi_ref[sl, pl.ds(r0, strip)]
            lo, hi = _bisect_step(l, lo, hi)
            lo, hi = _bisect_step(l, lo, hi)
            lo_ref[sl, pl.ds(r0, strip)] = lo
            hi_ref[sl, pl.ds(r0, strip)] = hi

            @pl.when(j == num_g - 1)
            def _exp_z(l=l, lo=lo, r0=r0):
                tf = _key_to_f32(lo)
                m = m_ref[sl, pl.ds(r0, strip)]
                e = jnp.where(l >= tf, jnp.exp(l - m), jnp.float32(0.0))
                z = jnp.sum(e, axis=1, keepdims=True)
                acc_ref[sl, pl.ds(r0, strip), :] = e
                zinv_ref[sl, pl.ds(r0, strip)] = 1.0 / z


def _topk_softmax(x2d, w, temp, rb, gt):
    m, d = x2d.shape
    g = w.shape[0]
    num_g = g // gt
    ni = m // rb
    grid = (ni + 2, num_g)
    return pl.pallas_call(
        functools.partial(_kernel_body, ni, num_g, gt),
        grid=grid,
        in_specs=[
            pl.BlockSpec(memory_space=pltpu.SMEM),
            pl.BlockSpec((rb, d), lambda i, j: (jnp.minimum(i, ni - 1), 0)),
            pl.BlockSpec((gt, d), lambda i, j: (j, 0)),
        ],
        out_specs=pl.BlockSpec(
            (rb, gt), lambda i, j: (jnp.maximum(i - 2, 0), j)),
        out_shape=jax.ShapeDtypeStruct((m, g), jnp.float32),
        scratch_shapes=[
            pltpu.VMEM((2, rb, g), jnp.float32),
            pltpu.VMEM((2, rb, 1), jnp.int32),
            pltpu.VMEM((2, rb, 1), jnp.int32),
            pltpu.VMEM((2, rb, 1), jnp.float32),
            pltpu.VMEM((2, rb, 1), jnp.float32),
        ],
        compiler_params=pltpu.CompilerParams(
            dimension_semantics=("arbitrary", "arbitrary"),
            vmem_limit_bytes=100 * 1024 * 1024,
        ),
    )(temp, x2d, w)


@jax.jit
def kernel(x, W, temperature):
    b, t, d = x.shape
    g = W.shape[0]
    temp = jnp.maximum(temperature, 0.1).reshape(1)
    out = _topk_softmax(x.reshape(b * t, d), W, temp, rb=512, gt=256)
    return out.reshape(b, t, g)
